# Initial kernel scaffold; baseline (speedup 1.0000x reference)
import jax
import jax.numpy as jnp
from jax.experimental import pallas as pl


def kernel(x, w1, b1, w2, b2):
    raise NotImplementedError("write your pallas kernel here")


# speedup vs baseline: 1.1865x; 1.1865x over previous
import functools

import jax
import jax.numpy as jnp
from jax.experimental import pallas as pl
from jax.experimental.pallas import tpu as pltpu

_LANE = 512


def _se_kernel(x_ref, w1_ref, b1_ref, w2_ref, b2_ref, o_ref,
               xs_ref, acc_ref, g_ref, *, hw, n_tiles, nb, c):
    phase = pl.program_id(1)
    j = pl.program_id(2)

    @pl.when(phase == 0)
    def _pool_phase():
        xb = x_ref[...]
        xs_ref[j] = xb

        @pl.when(j == 0)
        def _():
            acc_ref[...] = jnp.zeros_like(acc_ref)

        if hw % _LANE == 0:
            acc_ref[...] += jnp.sum(xb.astype(jnp.float32), axis=-1,
                                    keepdims=True)
        else:
            @pl.when(j < n_tiles - 1)
            def _():
                acc_ref[...] += jnp.sum(xb.astype(jnp.float32), axis=-1,
                                        keepdims=True)

            @pl.when(j == n_tiles - 1)
            def _():
                col = jax.lax.broadcasted_iota(jnp.int32, xb.shape, 1)
                xm = jnp.where(col < hw - (n_tiles - 1) * _LANE,
                               xb.astype(jnp.float32), 0.0)
                acc_ref[...] += jnp.sum(xm, axis=-1, keepdims=True)

        @pl.when(j == n_tiles - 1)
        def _gate():
            rows = nb * c
            s = acc_ref[...]
            m = s * w1_ref[...]
            bt = jax.lax.broadcasted_iota(jnp.int32, (nb, rows), 0)
            rt = jax.lax.broadcasted_iota(jnp.int32, (nb, rows), 1)
            ind_t = (rt // c == bt).astype(jnp.float32)
            y1 = jnp.dot(ind_t, m, preferred_element_type=jnp.float32)
            y1 = jnp.maximum(y1 + b1_ref[...], 0.0)
            y2 = jnp.dot(y1, w2_ref[...],
                         preferred_element_type=jnp.float32)
            g = jnp.tanh(y2 + b2_ref[...])
            ri = jax.lax.broadcasted_iota(jnp.int32, (rows, nb), 0)
            bi = jax.lax.broadcasted_iota(jnp.int32, (rows, nb), 1)
            ind = (ri // c == bi).astype(jnp.float32)
            gm = jnp.dot(ind, g, preferred_element_type=jnp.float32)
            ci = jax.lax.broadcasted_iota(jnp.int32, (rows, c), 0) % c
            cj = jax.lax.broadcasted_iota(jnp.int32, (rows, c), 1)
            g_ref[...] = jnp.sum(jnp.where(ci == cj, gm, 0.0),
                                 axis=1, keepdims=True)

    @pl.when(phase == 1)
    def _scale_phase():
        o_ref[...] = xs_ref[j] * g_ref[...].astype(o_ref.dtype)


def kernel(x, w1, b1, w2, b2):
    B, C, H, W = x.shape
    R = w1.shape[0]
    HW = H * W
    BC = B * C
    x2 = x.reshape(BC, HW)

    n_tiles = pl.cdiv(HW, _LANE)
    P = 4
    while B % P:
        P //= 2
    rows = BC // P
    nb = rows // C

    w1rep = jnp.tile(jnp.transpose(w1).astype(jnp.float32) / float(HW),
                     (nb, 1))
    w2t = jnp.transpose(w2).astype(jnp.float32)
    b1r = b1.reshape(1, R).astype(jnp.float32)
    b2r = b2.reshape(1, C).astype(jnp.float32)

    out2 = pl.pallas_call(
        functools.partial(_se_kernel, hw=HW, n_tiles=n_tiles, nb=nb, c=C),
        out_shape=jax.ShapeDtypeStruct((BC, HW), x.dtype),
        grid=(P, 2, n_tiles),
        in_specs=[
            pl.BlockSpec((rows, _LANE),
                         lambda p, ph, j: (p, jnp.where(ph == 0, j,
                                                        n_tiles - 1))),
            pl.BlockSpec((rows, R), lambda p, ph, j: (0, 0)),
            pl.BlockSpec((1, R), lambda p, ph, j: (0, 0)),
            pl.BlockSpec((R, C), lambda p, ph, j: (0, 0)),
            pl.BlockSpec((1, C), lambda p, ph, j: (0, 0)),
        ],
        out_specs=pl.BlockSpec((rows, _LANE),
                               lambda p, ph, j: (p, jnp.where(ph == 0, 0, j))),
        scratch_shapes=[
            pltpu.VMEM((n_tiles, rows, _LANE), x.dtype),
            pltpu.VMEM((rows, 1), jnp.float32),
            pltpu.VMEM((rows, 1), jnp.float32),
        ],
        compiler_params=pltpu.CompilerParams(
            dimension_semantics=("parallel", "arbitrary", "arbitrary")),
        cost_estimate=pl.CostEstimate(
            flops=2 * BC * HW + 4 * B * C * R, transcendentals=B * C,
            bytes_accessed=2 * BC * HW * x.dtype.itemsize + BC * 4),
    )(x2, w1rep, b1r, w2t, b2r)

    return out2.reshape(B, C, H, W)


# speedup vs baseline: 1.6414x; 1.3834x over previous
import functools

import jax
import jax.numpy as jnp
from jax.experimental import pallas as pl
from jax.experimental.pallas import tpu as pltpu


def _se_kernel(x_ref, w1_ref, b1_ref, w2_ref, b2_ref, o_ref):
    xb = x_ref[...]
    s1 = jnp.sum(xb.astype(jnp.float32), axis=1, keepdims=True)
    s = jnp.sum(s1, axis=2, keepdims=True)
    y1 = jnp.sum(s * w1_ref[...], axis=0, keepdims=True)
    y1 = jnp.maximum(y1 + b1_ref[...], 0.0)
    y2 = jnp.sum(w2_ref[...] * y1, axis=2, keepdims=True)
    g = jnp.tanh(y2 + b2_ref[...])
    o_ref[...] = xb * g.astype(o_ref.dtype)


def kernel(x, w1, b1, w2, b2):
    B, C, H, W = x.shape
    R = w1.shape[0]

    w1c = (jnp.transpose(w1).astype(jnp.float32) / float(H * W)
           ).reshape(C, 1, R)
    w2c = w2.astype(jnp.float32).reshape(C, 1, R)
    b1c = b1.astype(jnp.float32).reshape(1, 1, R)
    b2c = b2.astype(jnp.float32).reshape(C, 1, 1)

    return pl.pallas_call(
        _se_kernel,
        out_shape=jax.ShapeDtypeStruct((B, C, H, W), x.dtype),
        grid=(B,),
        in_specs=[
            pl.BlockSpec((None, C, H, W), lambda b: (b, 0, 0, 0)),
            pl.BlockSpec((C, 1, R), lambda b: (0, 0, 0)),
            pl.BlockSpec((1, 1, R), lambda b: (0, 0, 0)),
            pl.BlockSpec((C, 1, R), lambda b: (0, 0, 0)),
            pl.BlockSpec((C, 1, 1), lambda b: (0, 0, 0)),
        ],
        out_specs=pl.BlockSpec((None, C, H, W), lambda b: (b, 0, 0, 0)),
        compiler_params=pltpu.CompilerParams(
            dimension_semantics=("parallel",)),
        cost_estimate=pl.CostEstimate(
            flops=2 * B * C * H * W + 4 * B * C * R,
            transcendentals=B * C,
            bytes_accessed=2 * B * C * H * W * x.dtype.itemsize),
    )(x, w1c, b1c, w2c, b2c)


# speedup vs baseline: 10.9135x; 6.6489x over previous
import jax
import jax.numpy as jnp
from jax.experimental import pallas as pl
from jax.experimental.pallas import tpu as pltpu


def _se_kernel(x_ref, w1_ref, b1_ref, w2_ref, b2_ref, o_ref):
    xb = x_ref[...]
    s1 = jnp.sum(xb.astype(jnp.float32), axis=0)
    s = jnp.sum(s1, axis=0, keepdims=True)
    y1 = jnp.dot(s, w1_ref[...], preferred_element_type=jnp.float32)
    y1 = jnp.maximum(y1 + b1_ref[...], 0.0)
    y2 = jnp.dot(y1, w2_ref[...], preferred_element_type=jnp.float32)
    g = jnp.tanh(y2 + b2_ref[...])
    o_ref[...] = xb * g.astype(o_ref.dtype)


def kernel(x, w1, b1, w2, b2):
    B, C, H, W = x.shape
    R = w1.shape[0]

    xt = jnp.transpose(x, (0, 2, 3, 1))

    w1t = jnp.transpose(w1).astype(jnp.float32) / float(H * W)
    w2t = jnp.transpose(w2).astype(jnp.float32)
    b1r = b1.astype(jnp.float32).reshape(1, R)
    b2r = b2.astype(jnp.float32).reshape(1, C)

    out_t = pl.pallas_call(
        _se_kernel,
        out_shape=jax.ShapeDtypeStruct((B, H, W, C), x.dtype),
        grid=(B,),
        in_specs=[
            pl.BlockSpec((None, H, W, C), lambda b: (b, 0, 0, 0)),
            pl.BlockSpec((C, R), lambda b: (0, 0)),
            pl.BlockSpec((1, R), lambda b: (0, 0)),
            pl.BlockSpec((R, C), lambda b: (0, 0)),
            pl.BlockSpec((1, C), lambda b: (0, 0)),
        ],
        out_specs=pl.BlockSpec((None, H, W, C), lambda b: (b, 0, 0, 0)),
        compiler_params=pltpu.CompilerParams(
            dimension_semantics=("parallel",)),
        cost_estimate=pl.CostEstimate(
            flops=2 * B * C * H * W + 4 * B * C * R,
            transcendentals=B * C,
            bytes_accessed=2 * B * C * H * W * x.dtype.itemsize),
    )(xt, w1t, b1r, w2t, b2r)

    return jnp.transpose(out_t, (0, 3, 1, 2))


# speedup vs baseline: 11.5254x; 1.0561x over previous
import jax
import jax.numpy as jnp
from jax.experimental import pallas as pl
from jax.experimental.pallas import tpu as pltpu


_IMGS = 2


def _se_kernel(x_ref, w1_ref, b1_ref, w2_ref, b2_ref, o_ref):
    for i in range(_IMGS):
        xb = x_ref[i]
        s1 = jnp.sum(xb.astype(jnp.float32), axis=0)
        s = jnp.sum(s1, axis=0, keepdims=True)
        y1 = jnp.dot(s, w1_ref[...], preferred_element_type=jnp.float32)
        y1 = jnp.maximum(y1 + b1_ref[...], 0.0)
        y2 = jnp.dot(y1, w2_ref[...], preferred_element_type=jnp.float32)
        g = jnp.tanh(y2 + b2_ref[...])
        o_ref[i] = xb * g.astype(o_ref.dtype)


def kernel(x, w1, b1, w2, b2):
    B, C, H, W = x.shape
    R = w1.shape[0]

    xt = jnp.transpose(x, (0, 2, 3, 1))

    w1t = jnp.transpose(w1).astype(jnp.float32) / float(H * W)
    w2t = jnp.transpose(w2).astype(jnp.float32)
    b1r = b1.astype(jnp.float32).reshape(1, R)
    b2r = b2.astype(jnp.float32).reshape(1, C)

    out_t = pl.pallas_call(
        _se_kernel,
        out_shape=jax.ShapeDtypeStruct((B, H, W, C), x.dtype),
        grid=(B // _IMGS,),
        in_specs=[
            pl.BlockSpec((_IMGS, H, W, C), lambda b: (b, 0, 0, 0)),
            pl.BlockSpec((C, R), lambda b: (0, 0)),
            pl.BlockSpec((1, R), lambda b: (0, 0)),
            pl.BlockSpec((R, C), lambda b: (0, 0)),
            pl.BlockSpec((1, C), lambda b: (0, 0)),
        ],
        out_specs=pl.BlockSpec((_IMGS, H, W, C), lambda b: (b, 0, 0, 0)),
        compiler_params=pltpu.CompilerParams(
            dimension_semantics=("parallel",)),
        cost_estimate=pl.CostEstimate(
            flops=2 * B * C * H * W + 4 * B * C * R,
            transcendentals=B * C,
            bytes_accessed=2 * B * C * H * W * x.dtype.itemsize),
    )(xt, w1t, b1r, w2t, b2r)

    return jnp.transpose(out_t, (0, 3, 1, 2))


# speedup vs baseline: 11.8421x; 1.0275x over previous
import jax
import jax.numpy as jnp
from jax.experimental import pallas as pl
from jax.experimental.pallas import tpu as pltpu


_IMGS = 4


def _se_kernel(x_ref, w1_ref, b1_ref, w2_ref, b2_ref, o_ref):
    for i in range(_IMGS):
        xb = x_ref[i]
        s1 = jnp.sum(xb.astype(jnp.float32), axis=0)
        s = jnp.sum(s1, axis=0, keepdims=True)
        y1 = jnp.dot(s, w1_ref[...], preferred_element_type=jnp.float32)
        y1 = jnp.maximum(y1 + b1_ref[...], 0.0)
        y2 = jnp.dot(y1, w2_ref[...], preferred_element_type=jnp.float32)
        g = jnp.tanh(y2 + b2_ref[...])
        o_ref[i] = xb * g.astype(o_ref.dtype)


def kernel(x, w1, b1, w2, b2):
    B, C, H, W = x.shape
    R = w1.shape[0]

    xt = jnp.transpose(x, (0, 2, 3, 1))

    w1t = jnp.transpose(w1).astype(jnp.float32) / float(H * W)
    w2t = jnp.transpose(w2).astype(jnp.float32)
    b1r = b1.astype(jnp.float32).reshape(1, R)
    b2r = b2.astype(jnp.float32).reshape(1, C)

    out_t = pl.pallas_call(
        _se_kernel,
        out_shape=jax.ShapeDtypeStruct((B, H, W, C), x.dtype),
        grid=(B // _IMGS,),
        in_specs=[
            pl.BlockSpec((_IMGS, H, W, C), lambda b: (b, 0, 0, 0)),
            pl.BlockSpec((C, R), lambda b: (0, 0)),
            pl.BlockSpec((1, R), lambda b: (0, 0)),
            pl.BlockSpec((R, C), lambda b: (0, 0)),
            pl.BlockSpec((1, C), lambda b: (0, 0)),
        ],
        out_specs=pl.BlockSpec((_IMGS, H, W, C), lambda b: (b, 0, 0, 0)),
        compiler_params=pltpu.CompilerParams(
            dimension_semantics=("parallel",)),
        cost_estimate=pl.CostEstimate(
            flops=2 * B * C * H * W + 4 * B * C * R,
            transcendentals=B * C,
            bytes_accessed=2 * B * C * H * W * x.dtype.itemsize),
    )(xt, w1t, b1r, w2t, b2r)

    return jnp.transpose(out_t, (0, 3, 1, 2))


# speedup vs baseline: 12.0877x; 1.0207x over previous
import functools

import jax
import jax.numpy as jnp
from jax.experimental import pallas as pl
from jax.experimental.pallas import tpu as pltpu


_IMGS = 4

_DN_T = (((1,), (1,)), ((), ()))


def _se_kernel(x_ref, w1_ref, b1_ref, w2_ref, b2_ref, o_ref, *, inv_hw):
    for i in range(_IMGS):
        xb = x_ref[i]
        s1 = jnp.sum(xb.astype(jnp.float32), axis=0)
        s = jnp.sum(s1, axis=0, keepdims=True)
        p = s * inv_hw
        y1 = jax.lax.dot_general(p, w1_ref[...], _DN_T,
                                 preferred_element_type=jnp.float32)
        y1 = jnp.maximum(y1 + b1_ref[...], 0.0)
        y2 = jnp.dot(y1, w2_ref[...], preferred_element_type=jnp.float32)
        g = jnp.tanh(y2 + b2_ref[...])
        o_ref[i] = xb * g.astype(o_ref.dtype)


def kernel(x, w1, b1, w2, b2):
    B, C, H, W = x.shape
    R = w1.shape[0]

    xt = jnp.transpose(x, (0, 2, 3, 1))

    w2t = jnp.transpose(w2)
    b1r = b1.astype(jnp.float32).reshape(1, R)
    b2r = b2.astype(jnp.float32).reshape(1, C)

    out_t = pl.pallas_call(
        functools.partial(_se_kernel, inv_hw=1.0 / float(H * W)),
        out_shape=jax.ShapeDtypeStruct((B, H, W, C), x.dtype),
        grid=(B // _IMGS,),
        in_specs=[
            pl.BlockSpec((_IMGS, H, W, C), lambda b: (b, 0, 0, 0)),
            pl.BlockSpec((R, C), lambda b: (0, 0)),
            pl.BlockSpec((1, R), lambda b: (0, 0)),
            pl.BlockSpec((R, C), lambda b: (0, 0)),
            pl.BlockSpec((1, C), lambda b: (0, 0)),
        ],
        out_specs=pl.BlockSpec((_IMGS, H, W, C), lambda b: (b, 0, 0, 0)),
        compiler_params=pltpu.CompilerParams(
            dimension_semantics=("parallel",)),
        cost_estimate=pl.CostEstimate(
            flops=2 * B * C * H * W + 4 * B * C * R,
            transcendentals=B * C,
            bytes_accessed=2 * B * C * H * W * x.dtype.itemsize),
    )(xt, w1, b1r, w2t, b2r)

    return jnp.transpose(out_t, (0, 3, 1, 2))
